# Initial kernel scaffold; baseline (speedup 1.0000x reference)
import jax
import jax.numpy as jnp
from jax.experimental import pallas as pl


def kernel(text, text_lengths, emb, W1, b1, W2, b2):
    raise NotImplementedError("write your pallas kernel here")


# speedup vs baseline: 9.8268x; 9.8268x over previous
import functools

import jax
import jax.numpy as jnp
from jax import lax
from jax.experimental import pallas as pl
from jax.experimental.pallas import tpu as pltpu
from jax.experimental.pallas import tpu_sc as plsc

DP = 16
NC, NS, LN = 2, 16, 16


def _proj_body(emb_ref, wp_ref, out_ref):
    out_ref[...] = jnp.dot(emb_ref[...], wp_ref[...],
                           preferred_element_type=jnp.float32)


def _project(emb, wp):
    V, H = emb.shape
    blk = 8000
    assert V % blk == 0
    return pl.pallas_call(
        _proj_body,
        grid=(V // blk,),
        in_specs=[pl.BlockSpec((blk, H), lambda i: (i, 0)),
                  pl.BlockSpec((H, DP), lambda i: (0, 0))],
        out_specs=pl.BlockSpec((blk, DP), lambda i: (i, 0)),
        out_shape=jax.ShapeDtypeStruct((V, DP), jnp.float32),
    )(emb, wp)


@functools.cache
def _make_sc(B, L):
    NW = NC * NS
    rows_per_w = B // NW
    G = 16
    CH = G * L
    n_chunks = rows_per_w // G
    assert B % NW == 0 and rows_per_w % G == 0 and L % 8 == 0
    mesh = plsc.VectorSubcoreMesh(core_axis_name="c", subcore_axis_name="s")

    @functools.partial(
        pl.kernel,
        out_type=jax.ShapeDtypeStruct((B, DP), jnp.float32),
        mesh=mesh,
        compiler_params=pltpu.CompilerParams(use_tc_tiling_on_sc=False),
        scratch_types=[
            pltpu.VMEM((CH,), jnp.int32),
            pltpu.VMEM((CH, DP), jnp.float32),
            pltpu.VMEM((G, DP), jnp.float32),
            pltpu.VMEM((LN,), jnp.float32),
            pltpu.SemaphoreType.DMA,
        ],
    )
    def sc(text_hbm, p_hbm, bc_hbm, out_hbm, idx_v, rows_v, pooled_v, bc_v,
           sem):
        wid = lax.axis_index("s") * NC + lax.axis_index("c")
        base_row = wid * rows_per_w
        pltpu.sync_copy(bc_hbm, bc_v)
        bc_vec = bc_v[...]

        def chunk_body(c, _):
            row0 = base_row + c * G
            pltpu.sync_copy(text_hbm.at[pl.ds(row0 * L, CH)], idx_v)
            pltpu.async_copy(p_hbm.at[idx_v], rows_v, sem).wait()

            def row_body(r, _):
                def acc_body(i, acc):
                    b = r * L + i * 8
                    s01 = rows_v[b] + rows_v[b + 1]
                    s23 = rows_v[b + 2] + rows_v[b + 3]
                    s45 = rows_v[b + 4] + rows_v[b + 5]
                    s67 = rows_v[b + 6] + rows_v[b + 7]
                    return acc + ((s01 + s23) + (s45 + s67))

                pooled_v[r] = lax.fori_loop(0, L // 8, acc_body, bc_vec)
                return 0

            lax.fori_loop(0, G, row_body, 0)
            pltpu.sync_copy(pooled_v, out_hbm.at[pl.ds(row0, G)])
            return 0

        lax.fori_loop(0, n_chunks, chunk_body, 0)

    return sc


def kernel(text, text_lengths, emb, W1, b1, W2, b2):
    B, L = text.shape
    V, H = emb.shape
    ncls = W2.shape[1]
    wc = (W1 @ W2) * (1.0 / L)
    wp = jnp.zeros((H, DP), jnp.float32).at[:, :ncls].set(wc)
    bc = jnp.zeros((DP,), jnp.float32).at[:ncls].set(b1 @ W2 + b2)
    p = _project(emb, wp)
    text_flat = text.reshape(-1).astype(jnp.int32)
    out16 = _make_sc(B, L)(text_flat, p, bc)
    return out16[:, :ncls]


# speedup vs baseline: 14.1321x; 1.4381x over previous
import functools

import jax
import jax.numpy as jnp
from jax import lax
from jax.experimental import pallas as pl
from jax.experimental.pallas import tpu as pltpu
from jax.experimental.pallas import tpu_sc as plsc

DP = 16
NC, NS, LN = 2, 16, 16


def _proj_body(emb_ref, wbig_ref, out_ref):
    out_ref[...] = jnp.dot(emb_ref[...], wbig_ref[...],
                           preferred_element_type=jnp.float32)


def _project(emb, wp):
    V, H = emb.shape
    pack = 128 // DP
    rows = V // pack
    blk = 5000
    assert rows % blk == 0
    wbig = jnp.einsum('ij,kl->ikjl', jnp.eye(pack, dtype=jnp.float32),
                      wp).reshape(pack * H, pack * DP)
    out = pl.pallas_call(
        _proj_body,
        grid=(rows // blk,),
        in_specs=[pl.BlockSpec((blk, pack * H), lambda i: (i, 0)),
                  pl.BlockSpec((pack * H, pack * DP), lambda i: (0, 0))],
        out_specs=pl.BlockSpec((blk, pack * DP), lambda i: (i, 0)),
        out_shape=jax.ShapeDtypeStruct((rows, pack * DP), jnp.float32),
    )(emb.reshape(rows, pack * H), wbig)
    return out.reshape(V, DP)


@functools.cache
def _make_sc(B, L):
    NW = NC * NS
    rows_per_w = B // NW
    G = 16
    CH = G * L
    n_chunks = rows_per_w // G
    assert B % NW == 0 and rows_per_w % G == 0 and L % 8 == 0
    mesh = plsc.VectorSubcoreMesh(core_axis_name="c", subcore_axis_name="s")

    @functools.partial(
        pl.kernel,
        out_type=jax.ShapeDtypeStruct((B, DP), jnp.float32),
        mesh=mesh,
        compiler_params=pltpu.CompilerParams(use_tc_tiling_on_sc=False),
        scratch_types=[
            pltpu.VMEM((CH,), jnp.int32),
            pltpu.VMEM((CH, DP), jnp.float32),
            pltpu.VMEM((G, DP), jnp.float32),
            pltpu.VMEM((LN,), jnp.float32),
            pltpu.SemaphoreType.DMA,
        ],
    )
    def sc(text_hbm, p_hbm, bc_hbm, out_hbm, idx_v, rows_v, pooled_v, bc_v,
           sem):
        wid = lax.axis_index("s") * NC + lax.axis_index("c")
        base_row = wid * rows_per_w
        pltpu.sync_copy(bc_hbm, bc_v)
        bc_vec = bc_v[...]

        def chunk_body(c, _):
            row0 = base_row + c * G
            pltpu.sync_copy(text_hbm.at[pl.ds(row0 * L, CH)], idx_v)
            pltpu.async_copy(p_hbm.at[idx_v], rows_v, sem).wait()

            def row_body(r, _):
                def acc_body(i, acc):
                    b = r * L + i * 8
                    s01 = rows_v[b] + rows_v[b + 1]
                    s23 = rows_v[b + 2] + rows_v[b + 3]
                    s45 = rows_v[b + 4] + rows_v[b + 5]
                    s67 = rows_v[b + 6] + rows_v[b + 7]
                    return acc + ((s01 + s23) + (s45 + s67))

                pooled_v[r] = lax.fori_loop(0, L // 8, acc_body, bc_vec)
                return 0

            lax.fori_loop(0, G, row_body, 0)
            pltpu.sync_copy(pooled_v, out_hbm.at[pl.ds(row0, G)])
            return 0

        lax.fori_loop(0, n_chunks, chunk_body, 0)

    return sc


def kernel(text, text_lengths, emb, W1, b1, W2, b2):
    B, L = text.shape
    V, H = emb.shape
    ncls = W2.shape[1]
    wc = (W1 @ W2) * (1.0 / L)
    wp = jnp.zeros((H, DP), jnp.float32).at[:, :ncls].set(wc)
    bc = jnp.zeros((DP,), jnp.float32).at[:ncls].set(b1 @ W2 + b2)
    p = _project(emb, wp)
    text_flat = text.reshape(-1).astype(jnp.int32)
    out16 = _make_sc(B, L)(text_flat, p, bc)
    return out16[:, :ncls]
